# Initial kernel scaffold; baseline (speedup 1.0000x reference)
import jax
import jax.numpy as jnp
from jax.experimental import pallas as pl


def kernel(features, edge_index, W_msg, b_msg, Wz, Uz, bz, Wr, Ur, br, Wh, Uh, bh):
    raise NotImplementedError("write your pallas kernel here")


# speedup vs baseline: 6.2846x; 6.2846x over previous
import functools

import jax
import jax.numpy as jnp
from jax import lax
from jax.experimental import pallas as pl
from jax.experimental.pallas import tpu as pltpu
from jax.experimental.pallas import tpu_sc as plsc

N = 10000
E = 320000
D = 128
NUM_LAYERS = 2

NC = 2
NS = 16
CH = 128
NCHUNKS = E // CH
N_PAD = 10240
ROWS_PER_SUB = N_PAD // NS

_mesh = plsc.VectorSubcoreMesh(core_axis_name="c", subcore_axis_name="s")


@functools.partial(
    pl.kernel,
    mesh=_mesh,
    out_type=jax.ShapeDtypeStruct((NC, N_PAD, D), jnp.float32),
    scratch_types=[
        pltpu.VMEM((CH,), jnp.int32),
        pltpu.VMEM((CH,), jnp.int32),
        pltpu.VMEM((CH, D), jnp.float32),
        pltpu.VMEM_SHARED((N_PAD, D), jnp.float32),
        pltpu.SemaphoreType.DMA,
    ],
)
def _sc_segment_sum(m_hbm, src_hbm, dst_hbm, zero_hbm, out_hbm,
                    src_v, dst_v, rows_v, agg_sh, sem):
    c = lax.axis_index("c")
    s = lax.axis_index("s")
    wid = s * NC + c
    row0 = s * ROWS_PER_SUB

    pltpu.sync_copy(zero_hbm.at[pl.ds(row0, ROWS_PER_SUB)],
                    agg_sh.at[pl.ds(row0, ROWS_PER_SUB)])
    plsc.subcore_barrier()

    @pl.loop(wid, NCHUNKS, step=NC * NS)
    def _(chunk):
        base = chunk * CH
        pltpu.sync_copy(src_hbm.at[pl.ds(base, CH)], src_v)
        pltpu.sync_copy(dst_hbm.at[pl.ds(base, CH)], dst_v)
        pltpu.async_copy(m_hbm.at[src_v], rows_v, sem).wait()
        pltpu.sync_copy(rows_v, agg_sh.at[dst_v], add=True)

    plsc.subcore_barrier()
    pltpu.sync_copy(agg_sh.at[pl.ds(row0, ROWS_PER_SUB)],
                    out_hbm.at[c, pl.ds(row0, ROWS_PER_SUB)])


BLK = 2000


def _dot(a, b):
    return jnp.dot(a, b, preferred_element_type=jnp.float32)


def _linear_body(h_ref, w_ref, b_ref, o_ref):
    o_ref[...] = _dot(h_ref[...], w_ref[...]) + b_ref[...]


def _tc_linear(h, w, b):
    return pl.pallas_call(
        _linear_body,
        grid=(N // BLK,),
        in_specs=[
            pl.BlockSpec((BLK, D), lambda i: (i, 0)),
            pl.BlockSpec((D, D), lambda i: (0, 0)),
            pl.BlockSpec((1, D), lambda i: (0, 0)),
        ],
        out_specs=pl.BlockSpec((BLK, D), lambda i: (i, 0)),
        out_shape=jax.ShapeDtypeStruct((N, D), jnp.float32),
    )(h, w, b.reshape(1, D))


def _gru_body(h_ref, agg_ref, Wz_ref, Uz_ref, bz_ref, Wr_ref, Ur_ref, br_ref,
              Wh_ref, Uh_ref, bh_ref, Wm_ref, bm_ref, h_out_ref, m_out_ref):
    agg = agg_ref[0] + agg_ref[1]
    h = h_ref[...]
    z = jax.nn.sigmoid(_dot(agg, Wz_ref[...]) + _dot(h, Uz_ref[...])
                       + bz_ref[...])
    r = jax.nn.sigmoid(_dot(agg, Wr_ref[...]) + _dot(h, Ur_ref[...])
                       + br_ref[...])
    h_t = jnp.tanh(_dot(agg, Wh_ref[...]) + _dot(r * h, Uh_ref[...])
                   + bh_ref[...])
    h_new = (1.0 - z) * h + z * h_t
    h_out_ref[...] = h_new
    if m_out_ref is not None:
        m_out_ref[...] = _dot(h_new, Wm_ref[...]) + bm_ref[...]


def _tc_gru(h, agg2, Wz, Uz, bz, Wr, Ur, br, Wh, Uh, bh, Wm, bm,
            compute_m):
    n_out = 2 if compute_m else 1
    body = _gru_body if compute_m else (
        lambda *refs: _gru_body(*refs, None))
    wspec = pl.BlockSpec((D, D), lambda i: (0, 0))
    bspec = pl.BlockSpec((1, D), lambda i: (0, 0))
    rowspec = pl.BlockSpec((BLK, D), lambda i: (i, 0))
    out = pl.pallas_call(
        body,
        grid=(N // BLK,),
        in_specs=[
            rowspec,
            pl.BlockSpec((NC, BLK, D), lambda i: (0, i, 0)),
            wspec, wspec, bspec,
            wspec, wspec, bspec,
            wspec, wspec, bspec,
            wspec, bspec,
        ],
        out_specs=[rowspec] * n_out,
        out_shape=[jax.ShapeDtypeStruct((N, D), jnp.float32)] * n_out,
    )(h, agg2, Wz, Uz, bz.reshape(1, D), Wr, Ur, br.reshape(1, D),
      Wh, Uh, bh.reshape(1, D), Wm, bm.reshape(1, D))
    return out


@jax.jit
def kernel(features, edge_index, W_msg, b_msg, Wz, Uz, bz, Wr, Ur, br,
           Wh, Uh, bh):
    src = edge_index[0]
    dst = edge_index[1]
    zeros = jnp.zeros((N_PAD, D), jnp.float32)
    h = features
    m = _tc_linear(features, W_msg, b_msg)
    for layer in range(NUM_LAYERS):
        agg2 = _sc_segment_sum(m, src, dst, zeros)
        compute_m = layer < NUM_LAYERS - 1
        out = _tc_gru(h, agg2, Wz, Uz, bz, Wr, Ur, br, Wh, Uh, bh,
                      W_msg, b_msg, compute_m)
        if compute_m:
            h, m = out
        else:
            (h,) = out
    return h
